# baseline (speedup 1.0000x reference)
import jax
import jax.numpy as jnp
from jax import lax
from jax.experimental import pallas as pl
from jax.experimental.pallas import tpu as pltpu
from jax.experimental.pallas import tpu_sc as plsc

N_BATCH = 16
C_IN = 256
C_OUT = 256
HW = 4096
P_TOTAL = N_BATCH * HW
TP = 8192
NTP = P_TOTAL // TP
MROWS = 2
SC_CHUNK = 2048

SC_CORES = 2
SC_SUBCORES = 16
N_ANCHOR = 5
LANES = 16


def _sc_mask_kernel(lab_hbm, out_hbm, lab_v, out_v):
    wid = lax.axis_index("s") * SC_CORES + lax.axis_index("c")
    n = wid // (HW // SC_CHUNK)
    off = (wid % (HW // SC_CHUNK)) * SC_CHUNK
    pltpu.sync_copy(lab_hbm.at[n, :, pl.ds(off, SC_CHUNK)], lab_v)

    def body(i, carry):
        base = pl.multiple_of(i * LANES, LANES)
        one = jnp.int32(1)
        zero = jnp.int32(0)
        acc_eq = jnp.zeros((LANES,), jnp.int32)
        acc_ge = jnp.zeros((LANES,), jnp.int32)
        for a_idx in range(N_ANCHOR):
            l = lab_v[a_idx, pl.ds(base, LANES)]
            acc_eq += one - jnp.minimum(jnp.abs(l - one), one)
            acc_ge += one - jnp.minimum(jnp.maximum(-l, zero), one)
        pos_i = jnp.minimum(acc_eq, one)
        neg_i = jnp.minimum(acc_ge, one) * (one - pos_i)
        out_v[0, pl.ds(base, LANES)] = pos_i.astype(jnp.float32)
        out_v[1, pl.ds(base, LANES)] = neg_i.astype(jnp.float32)
        return carry

    lax.fori_loop(0, SC_CHUNK // LANES, body, 0)
    t = wid // (TP // SC_CHUNK)
    col = (wid % (TP // SC_CHUNK)) * SC_CHUNK
    pltpu.sync_copy(out_v, out_hbm.at[t, :, pl.ds(col, SC_CHUNK)])


def _build_masks(lab):
    mesh = plsc.VectorSubcoreMesh(core_axis_name="c", subcore_axis_name="s")
    return pl.kernel(
        _sc_mask_kernel,
        mesh=mesh,
        out_type=jax.ShapeDtypeStruct((NTP, MROWS, TP), jnp.float32),
        scratch_types=[
            pltpu.VMEM((N_ANCHOR, SC_CHUNK), jnp.int32),
            pltpu.VMEM((MROWS, SC_CHUNK), jnp.float32),
        ],
    )(lab)


def _fold(v):
    w = v.shape[1]
    while w > 256:
        w //= 2
        v = v[:, :w] + v[:, w:]
    return v


def _dot_t(a, b):
    return lax.dot_general(a, b, (((1,), (1,)), ((), ())),
                           preferred_element_type=jnp.float32)


def _tc_body(x_ref, w_ref, m_ref, gb_ref, out_ref,
             ybf_s, asum, asq, macc, cnt, a_s, b_s):
    ph = pl.program_id(0)
    t = pl.program_id(1)

    @pl.when((ph == 0) & (t == 0))
    def _init():
        asum[...] = jnp.zeros_like(asum)
        asq[...] = jnp.zeros_like(asq)
        macc[...] = jnp.zeros_like(macc)
        cnt[...] = jnp.zeros_like(cnt)

    @pl.when(ph == 0)
    def _phase0():
        xbf = x_ref[...].astype(jnp.bfloat16)
        wbf = w_ref[...].astype(jnp.bfloat16)
        ybf = _dot_t(xbf, wbf).astype(jnp.bfloat16)
        ybf_s[pl.ds(t * TP, TP), :] = ybf
        o1 = jnp.ones((1, TP), jnp.bfloat16)
        asum[...] += jnp.dot(o1, ybf, preferred_element_type=jnp.float32)
        asq[...] += jnp.dot(o1, ybf * ybf, preferred_element_type=jnp.float32)

    @pl.when((ph == 0) & (t == NTP - 1))
    def _mid():
        inv_p = jnp.float32(1.0 / P_TOTAL)
        mean = asum[...] * inv_p
        var = asq[...] * inv_p - mean * mean
        a = gb_ref[0:1, :] * lax.rsqrt(var + jnp.float32(1e-5))
        a_s[...] = a
        b_s[...] = gb_ref[1:2, :] - a * mean

    @pl.when(ph == 1)
    def _phase1():
        yv = ybf_s[pl.ds(t * TP, TP), :]
        ab = a_s[...].astype(jnp.bfloat16)
        bb = b_s[...].astype(jnp.bfloat16)
        z = jnp.maximum(ab * yv + bb, jnp.bfloat16(0.0))
        zz = jnp.concatenate([z, z * z], axis=1)
        m2 = m_ref[0]
        macc[...] += jnp.dot(m2.astype(jnp.bfloat16), zz,
                             preferred_element_type=jnp.float32)
        cnt[...] += _fold(m2)

    @pl.when((ph == 1) & (t == NTP - 1))
    def _fin():
        n_pos = jnp.sum(cnt[0:1, :])
        n_neg = jnp.sum(cnt[1:2, :])
        mu_p = macc[0:1, :C_OUT] / n_pos
        mu_n = macc[1:2, :C_OUT] / n_neg
        ss_p = macc[0:1, C_OUT:]
        ss_n = macc[1:2, C_OUT:]
        var_p = (ss_p - n_pos * mu_p * mu_p) / (n_pos - jnp.float32(1.0))
        var_n = (ss_n - n_neg * mu_n * mu_n) / (n_neg - jnp.float32(1.0))
        out_ref[...] = jnp.concatenate(
            [mu_p, mu_n, jnp.sqrt(var_p), jnp.sqrt(var_n)], axis=0)


def _tc_call(xt, w, masks, gb):
    return pl.pallas_call(
        _tc_body,
        grid=(2, NTP),
        in_specs=[
            pl.BlockSpec((TP, C_IN), lambda ph, t: (t * (1 - ph), 0)),
            pl.BlockSpec((C_OUT, C_IN), lambda ph, t: (0, 0)),
            pl.BlockSpec((1, MROWS, TP), lambda ph, t: (t, 0, 0)),
            pl.BlockSpec((2, C_OUT), lambda ph, t: (0, 0)),
        ],
        out_specs=pl.BlockSpec((4, C_OUT), lambda ph, t: (0, 0)),
        out_shape=jax.ShapeDtypeStruct((4, C_OUT), jnp.float32),
        scratch_shapes=[
            pltpu.VMEM((P_TOTAL, C_OUT), jnp.bfloat16),
            pltpu.VMEM((1, C_OUT), jnp.float32),
            pltpu.VMEM((1, C_OUT), jnp.float32),
            pltpu.VMEM((MROWS, 2 * C_OUT), jnp.float32),
            pltpu.VMEM((MROWS, C_OUT), jnp.float32),
            pltpu.VMEM((1, C_OUT), jnp.float32),
            pltpu.VMEM((1, C_OUT), jnp.float32),
        ],
    )(xt, w, masks, gb)


def kernel(hidden_map, label_cls, W, gamma, beta):
    s = hidden_map.shape
    xt = jnp.transpose(hidden_map, (0, 2, 3, 1)).reshape(P_TOTAL, C_IN)
    lab = label_cls.astype(jnp.int32).reshape(N_BATCH, N_ANCHOR, HW)
    masks = _build_masks(lab)
    gb = jnp.stack([gamma, beta])
    out = _tc_call(xt, W, masks, gb)
    latents = out.reshape(1, 4 * C_OUT, 1, 1)
    return (latents, 0.0, s)


# speedup vs baseline: 1.0055x; 1.0055x over previous
import jax
import jax.numpy as jnp
from jax import lax
from jax.experimental import pallas as pl
from jax.experimental.pallas import tpu as pltpu
from jax.experimental.pallas import tpu_sc as plsc

N_BATCH = 16
C_IN = 256
C_OUT = 256
HW = 4096
P_TOTAL = N_BATCH * HW
TP = 8192
NTP = P_TOTAL // TP
MROWS = 2
SC_CHUNK = 2048

SC_CORES = 2
SC_SUBCORES = 16
N_ANCHOR = 5
LANES = 16


def _sc_mask_kernel(lab_hbm, out_hbm, lab_v, out_v):
    wid = lax.axis_index("s") * SC_CORES + lax.axis_index("c")
    n = wid // (HW // SC_CHUNK)
    h0 = (wid % (HW // SC_CHUNK)) * (SC_CHUNK // 64)
    pltpu.sync_copy(lab_hbm.at[n, :, pl.ds(h0, SC_CHUNK // 64), :], lab_v)

    def body(i, carry):
        base = pl.multiple_of(i * LANES, LANES)
        one = jnp.int32(1)
        zero = jnp.int32(0)
        acc_eq = jnp.zeros((LANES,), jnp.int32)
        acc_ge = jnp.zeros((LANES,), jnp.int32)
        for a_idx in range(N_ANCHOR):
            l = lab_v[a_idx, lax.div(i, 4), pl.ds(lax.rem(i, 4) * LANES, LANES)]
            acc_eq += one - jnp.minimum(jnp.abs(l - one), one)
            acc_ge += one - jnp.minimum(jnp.maximum(-l, zero), one)
        pos_i = jnp.minimum(acc_eq, one)
        neg_i = jnp.minimum(acc_ge, one) * (one - pos_i)
        out_v[0, pl.ds(base, LANES)] = pos_i.astype(jnp.float32)
        out_v[1, pl.ds(base, LANES)] = neg_i.astype(jnp.float32)
        return carry

    lax.fori_loop(0, SC_CHUNK // LANES, body, 0)
    t = wid // (TP // SC_CHUNK)
    col = (wid % (TP // SC_CHUNK)) * SC_CHUNK
    pltpu.sync_copy(out_v, out_hbm.at[t, :, pl.ds(col, SC_CHUNK)])


def _build_masks(lab):
    mesh = plsc.VectorSubcoreMesh(core_axis_name="c", subcore_axis_name="s")
    return pl.kernel(
        _sc_mask_kernel,
        mesh=mesh,
        out_type=jax.ShapeDtypeStruct((NTP, MROWS, TP), jnp.float32),
        scratch_types=[
            pltpu.VMEM((N_ANCHOR, SC_CHUNK // 64, 64), jnp.int32),
            pltpu.VMEM((MROWS, SC_CHUNK), jnp.float32),
        ],
    )(lab)


def _fold(v):
    w = v.shape[1]
    while w > 256:
        w //= 2
        v = v[:, :w] + v[:, w:]
    return v


def _dot_t(a, b):
    return lax.dot_general(a, b, (((1,), (1,)), ((), ())),
                           preferred_element_type=jnp.float32)


def _tc_body(x_ref, w_ref, m_ref, gb_ref, out_ref,
             ybf_s, asum, asq, macc, cnt, a_s, b_s):
    ph = pl.program_id(0)
    t = pl.program_id(1)

    @pl.when((ph == 0) & (t == 0))
    def _init():
        asum[...] = jnp.zeros_like(asum)
        asq[...] = jnp.zeros_like(asq)
        macc[...] = jnp.zeros_like(macc)
        cnt[...] = jnp.zeros_like(cnt)

    @pl.when(ph == 0)
    def _phase0():
        xbf = x_ref[...].astype(jnp.bfloat16)
        wbf = w_ref[...].astype(jnp.bfloat16)
        ybf = _dot_t(xbf, wbf).astype(jnp.bfloat16)
        ybf_s[pl.ds(t * TP, TP), :] = ybf
        o1 = jnp.ones((1, TP), jnp.bfloat16)
        asum[...] += jnp.dot(o1, ybf, preferred_element_type=jnp.float32)
        asq[...] += jnp.dot(o1, ybf * ybf, preferred_element_type=jnp.float32)

    @pl.when((ph == 0) & (t == NTP - 1))
    def _mid():
        inv_p = jnp.float32(1.0 / P_TOTAL)
        mean = asum[...] * inv_p
        var = asq[...] * inv_p - mean * mean
        a = gb_ref[0:1, :] * lax.rsqrt(var + jnp.float32(1e-5))
        a_s[...] = a
        b_s[...] = gb_ref[1:2, :] - a * mean

    @pl.when(ph == 1)
    def _phase1():
        yv = ybf_s[pl.ds(t * TP, TP), :]
        ab = a_s[...].astype(jnp.bfloat16)
        bb = b_s[...].astype(jnp.bfloat16)
        z = jnp.maximum(ab * yv + bb, jnp.bfloat16(0.0))
        zz = jnp.concatenate([z, z * z], axis=1)
        m2 = m_ref[0]
        macc[...] += jnp.dot(m2.astype(jnp.bfloat16), zz,
                             preferred_element_type=jnp.float32)
        cnt[...] += _fold(m2)

    @pl.when((ph == 1) & (t == NTP - 1))
    def _fin():
        n_pos = jnp.sum(cnt[0:1, :])
        n_neg = jnp.sum(cnt[1:2, :])
        mu_p = macc[0:1, :C_OUT] / n_pos
        mu_n = macc[1:2, :C_OUT] / n_neg
        ss_p = macc[0:1, C_OUT:]
        ss_n = macc[1:2, C_OUT:]
        var_p = (ss_p - n_pos * mu_p * mu_p) / (n_pos - jnp.float32(1.0))
        var_n = (ss_n - n_neg * mu_n * mu_n) / (n_neg - jnp.float32(1.0))
        out_ref[...] = jnp.concatenate(
            [mu_p, mu_n, jnp.sqrt(var_p), jnp.sqrt(var_n)], axis=0)


def _tc_call(xt, w, masks, gb):
    return pl.pallas_call(
        _tc_body,
        grid=(2, NTP),
        in_specs=[
            pl.BlockSpec((TP, C_IN), lambda ph, t: (t * (1 - ph), 0)),
            pl.BlockSpec((C_OUT, C_IN), lambda ph, t: (0, 0)),
            pl.BlockSpec((1, MROWS, TP), lambda ph, t: (t, 0, 0)),
            pl.BlockSpec((2, C_OUT), lambda ph, t: (0, 0)),
        ],
        out_specs=pl.BlockSpec((4, C_OUT), lambda ph, t: (0, 0)),
        out_shape=jax.ShapeDtypeStruct((4, C_OUT), jnp.float32),
        scratch_shapes=[
            pltpu.VMEM((P_TOTAL, C_OUT), jnp.bfloat16),
            pltpu.VMEM((1, C_OUT), jnp.float32),
            pltpu.VMEM((1, C_OUT), jnp.float32),
            pltpu.VMEM((MROWS, 2 * C_OUT), jnp.float32),
            pltpu.VMEM((MROWS, C_OUT), jnp.float32),
            pltpu.VMEM((1, C_OUT), jnp.float32),
            pltpu.VMEM((1, C_OUT), jnp.float32),
        ],
    )(xt, w, masks, gb)


def kernel(hidden_map, label_cls, W, gamma, beta):
    s = hidden_map.shape
    xt = jnp.transpose(hidden_map, (0, 2, 3, 1)).reshape(P_TOTAL, C_IN)
    lab = label_cls.astype(jnp.int32)
    masks = _build_masks(lab)
    gb = jnp.stack([gamma, beta])
    out = _tc_call(xt, W, masks, gb)
    latents = out.reshape(1, 4 * C_OUT, 1, 1)
    return (latents, 0.0, s)
